# Initial kernel scaffold; baseline (speedup 1.0000x reference)
import jax
import jax.numpy as jnp
from jax.experimental import pallas as pl


def kernel(x, w1, b1, g1, be1, m1, v1, w2, b2, g2, be2, m2, v2, wp, bp):
    raise NotImplementedError("write your pallas kernel here")


# speedup vs baseline: 5.1588x; 5.1588x over previous
import functools

import jax
import jax.numpy as jnp
from jax.experimental import pallas as pl
from jax.experimental.pallas import tpu as pltpu


def _elu(v):
    return jnp.where(v > 0, v, jnp.exp(jnp.minimum(v, 0.0)) - 1.0)


def _fused_kernel(x_ref, wt_ref, a1_ref, c1_ref, w2_ref, a2_ref, c2_ref,
                  wp_ref, bp_ref, o_ref, *, w0s, tw, rw, C, tpad):
    Bb, H, T = x_ref.shape
    x2 = x_ref[...].astype(jnp.bfloat16).reshape(Bb * H, T)
    if tpad > 0:
        x2 = jnp.concatenate(
            [x2, jnp.zeros((Bb * H, tpad), jnp.bfloat16)], axis=1)
    for w0 in w0s:
        t0 = 5 * w0
        xs = x2[:, t0:t0 + rw]
        y = jnp.dot(xs, wt_ref[...],
                    preferred_element_type=jnp.float32)
        y = a1_ref[...] * y + c1_ref[...]
        y = _elu(y).astype(jnp.bfloat16)
        z = (y.reshape(Bb, H, tw, C)
              .transpose(0, 2, 1, 3)
              .reshape(Bb * tw, H * C))
        h = jnp.dot(z, w2_ref[...],
                    preferred_element_type=jnp.float32)
        h = a2_ref[...] * h + c2_ref[...]
        h = _elu(h).astype(jnp.bfloat16)
        out = jnp.dot(h, wp_ref[...],
                      preferred_element_type=jnp.float32) + bp_ref[...]
        o_ref[:, w0:w0 + tw, :] = out.reshape(Bb, tw, out.shape[-1])


def kernel(x, w1, b1, g1, be1, m1, v1, w2, b2, g2, be2, m2, v2, wp, bp):
    B, H, T = x.shape
    C, K1, PK, PS, eps = 40, 25, 51, 5, 1e-5
    E = wp.shape[0]
    Wo = T - K1 + 1
    Wp = (Wo - PK) // PS + 1
    BW = PK + K1 - 1

    cums = jnp.concatenate(
        [jnp.zeros((C, 1), jnp.float32), jnp.cumsum(w1, axis=1)], axis=1)
    d = jnp.arange(BW)
    hi = jnp.minimum(d, K1 - 1) + 1
    lo = jnp.maximum(d - (PK - 1), 0)
    band = ((cums[:, hi] - cums[:, lo]) / PK).T

    tw = min(16, Wp)
    rw = BW + PS * (tw - 1)
    r = jnp.arange(rw)[:, None]
    wr = jnp.arange(tw)[None, :]
    dd = r - PS * wr
    valid = (dd >= 0) & (dd < BW)
    wt = jnp.where(valid[:, :, None], band[jnp.clip(dd, 0, BW - 1)], 0.0)
    wt = wt.reshape(rw, tw * C).astype(jnp.bfloat16)

    a1 = g1 / jnp.sqrt(v1 + eps)
    c1 = be1 + a1 * (b1 - m1)
    a2 = g2 / jnp.sqrt(v2 + eps)
    c2 = be2 + a2 * (b2 - m2)
    a1b = jnp.tile(a1, tw)[None, :].astype(jnp.float32)
    c1b = jnp.tile(c1, tw)[None, :].astype(jnp.float32)
    a2r = a2[None, :].astype(jnp.float32)
    c2r = c2[None, :].astype(jnp.float32)
    w2t = jnp.transpose(w2, (2, 1, 0)).reshape(H * C, C).astype(jnp.bfloat16)
    wpt = jnp.transpose(wp).astype(jnp.bfloat16)
    bpr = bp[None, :].astype(jnp.float32)

    Bb = next(dv for dv in (8, 4, 2, 1) if B % dv == 0)
    nt = -(-Wp // tw)
    Wpad = nt * tw
    w0s = tuple(range(0, Wpad, tw))
    tpad = max(0, PS * w0s[-1] + rw - T)

    kern = functools.partial(_fused_kernel, w0s=w0s, tw=tw, rw=rw, C=C,
                             tpad=tpad)
    flops = 2 * B * nt * (H * rw * tw * C + tw * H * C * C + tw * C * E)
    out = pl.pallas_call(
        kern,
        out_shape=jax.ShapeDtypeStruct((B, Wpad, E), jnp.float32),
        grid=(B // Bb,),
        in_specs=[
            pl.BlockSpec((Bb, H, T), lambda i: (i, 0, 0)),
            pl.BlockSpec((rw, tw * C), lambda i: (0, 0)),
            pl.BlockSpec((1, tw * C), lambda i: (0, 0)),
            pl.BlockSpec((1, tw * C), lambda i: (0, 0)),
            pl.BlockSpec((H * C, C), lambda i: (0, 0)),
            pl.BlockSpec((1, C), lambda i: (0, 0)),
            pl.BlockSpec((1, C), lambda i: (0, 0)),
            pl.BlockSpec((C, E), lambda i: (0, 0)),
            pl.BlockSpec((1, E), lambda i: (0, 0)),
        ],
        out_specs=pl.BlockSpec((Bb, Wpad, E), lambda i: (i, 0, 0)),
        compiler_params=pltpu.CompilerParams(
            dimension_semantics=("parallel",),
            vmem_limit_bytes=64 * 1024 * 1024,
        ),
        cost_estimate=pl.CostEstimate(
            flops=int(flops),
            transcendentals=int(B * nt * tw * C * (H + 1)),
            bytes_accessed=int(4 * B * H * T + 4 * B * Wpad * E),
        ),
    )(x, wt, a1b, c1b, w2t, a2r, c2r, wpt, bpr)
    return out[:, :Wp, :]


# speedup vs baseline: 7.7435x; 1.5010x over previous
import functools

import jax
import jax.numpy as jnp
from jax.experimental import pallas as pl
from jax.experimental.pallas import tpu as pltpu


def _elu(v):
    return jnp.where(v > 0, v, jnp.exp(jnp.minimum(v, 0.0)) - 1.0)


def _fused_kernel(x_ref, wt_ref, a1_ref, c1_ref, w2_ref, a2_ref, c2_ref,
                  wp_ref, bp_ref, o_ref, *, w0s, tw, rw, C, tpad):
    Bb, H, T = x_ref.shape
    E = wp_ref.shape[1]
    x2 = x_ref[...].astype(jnp.bfloat16).reshape(Bb * H, T)
    if tpad > 0:
        x2 = jnp.concatenate(
            [x2, jnp.zeros((Bb * H, tpad), jnp.bfloat16)], axis=1)
    for w0 in w0s:
        t0 = 5 * w0
        xs = x2[:, t0:t0 + rw]
        y = jnp.dot(xs, wt_ref[...],
                    preferred_element_type=jnp.float32)
        yb = y.astype(jnp.bfloat16)
        yb = (a1_ref[...].astype(jnp.bfloat16) * yb
              + c1_ref[...].astype(jnp.bfloat16))
        yb = _elu(yb)
        q = jnp.swapaxes(yb.reshape(Bb, H, C * tw), 1, 2)
        hacc = jnp.zeros((Bb * tw, E), jnp.float32)
        for c in range(C):
            qc = q[:, c * tw:(c + 1) * tw, :].reshape(Bb * tw, H)
            hacc = jnp.dot(qc, w2_ref[c],
                           preferred_element_type=jnp.float32) + hacc
        h = a2_ref[...] * hacc + c2_ref[...]
        h = _elu(h).astype(jnp.bfloat16)
        out = jnp.dot(h, wp_ref[...],
                      preferred_element_type=jnp.float32) + bp_ref[...]
        o_ref[:, w0:w0 + tw, :] = out.reshape(Bb, tw, E)


def kernel(x, w1, b1, g1, be1, m1, v1, w2, b2, g2, be2, m2, v2, wp, bp):
    B, H, T = x.shape
    C, K1, PK, PS, eps = 40, 25, 51, 5, 1e-5
    E = wp.shape[0]
    Wo = T - K1 + 1
    Wp = (Wo - PK) // PS + 1
    BW = PK + K1 - 1

    cums = jnp.concatenate(
        [jnp.zeros((C, 1), jnp.float32), jnp.cumsum(w1, axis=1)], axis=1)
    d = jnp.arange(BW)
    hi = jnp.minimum(d, K1 - 1) + 1
    lo = jnp.maximum(d - (PK - 1), 0)
    band = ((cums[:, hi] - cums[:, lo]) / PK).T

    tw = min(16, Wp)
    rw = BW + PS * (tw - 1)
    r = jnp.arange(rw)[:, None]
    wr = jnp.arange(tw)[None, :]
    dd = r - PS * wr
    valid = (dd >= 0) & (dd < BW)
    wt = jnp.where(valid[:, :, None], band[jnp.clip(dd, 0, BW - 1)], 0.0)
    wt = jnp.transpose(wt, (0, 2, 1)).reshape(rw, C * tw).astype(jnp.bfloat16)

    a1 = g1 / jnp.sqrt(v1 + eps)
    c1 = be1 + a1 * (b1 - m1)
    a2 = g2 / jnp.sqrt(v2 + eps)
    c2 = be2 + a2 * (b2 - m2)
    a1b = jnp.repeat(a1, tw)[None, :].astype(jnp.float32)
    c1b = jnp.repeat(c1, tw)[None, :].astype(jnp.float32)
    a2r = a2[None, :].astype(jnp.float32)
    c2r = c2[None, :].astype(jnp.float32)
    w2hc = jnp.transpose(w2, (1, 2, 0)).astype(jnp.bfloat16)
    wpt = jnp.transpose(wp).astype(jnp.bfloat16)
    bpr = bp[None, :].astype(jnp.float32)

    Bb = next(dv for dv in (8, 4, 2, 1) if B % dv == 0)
    nt = -(-Wp // tw)
    Wpad = nt * tw
    w0s = tuple(range(0, Wpad, tw))
    tpad = max(0, PS * w0s[-1] + rw - T)

    kern = functools.partial(_fused_kernel, w0s=w0s, tw=tw, rw=rw, C=C,
                             tpad=tpad)
    flops = 2 * B * nt * (H * rw * tw * C + tw * H * C * E + tw * C * E)
    out = pl.pallas_call(
        kern,
        out_shape=jax.ShapeDtypeStruct((B, Wpad, E), jnp.float32),
        grid=(B // Bb,),
        in_specs=[
            pl.BlockSpec((Bb, H, T), lambda i: (i, 0, 0)),
            pl.BlockSpec((rw, C * tw), lambda i: (0, 0)),
            pl.BlockSpec((1, C * tw), lambda i: (0, 0)),
            pl.BlockSpec((1, C * tw), lambda i: (0, 0)),
            pl.BlockSpec((C, H, E), lambda i: (0, 0, 0)),
            pl.BlockSpec((1, E), lambda i: (0, 0)),
            pl.BlockSpec((1, E), lambda i: (0, 0)),
            pl.BlockSpec((C, E), lambda i: (0, 0)),
            pl.BlockSpec((1, E), lambda i: (0, 0)),
        ],
        out_specs=pl.BlockSpec((Bb, Wpad, E), lambda i: (i, 0, 0)),
        compiler_params=pltpu.CompilerParams(
            dimension_semantics=("parallel",),
            vmem_limit_bytes=64 * 1024 * 1024,
        ),
        cost_estimate=pl.CostEstimate(
            flops=int(flops),
            transcendentals=int(B * nt * tw * C * (H + 1)),
            bytes_accessed=int(4 * B * H * T + 4 * B * Wpad * E),
        ),
    )(x, wt, a1b, c1b, w2hc, a2r, c2r, wpt, bpr)
    return out[:, :Wp, :]


# speedup vs baseline: 9.0794x; 1.1725x over previous
import functools

import jax
import jax.numpy as jnp
from jax.experimental import pallas as pl
from jax.experimental.pallas import tpu as pltpu


def _elu(v):
    return jnp.where(v > 0, v, jnp.exp(jnp.minimum(v, 0.0)) - 1.0)


def _fused_kernel(x_ref, wt_ref, a1_ref, c1_ref, w2_ref, a2_ref, c2_ref,
                  wp_ref, bp_ref, o_ref, *, w0s, tw, rw, C, tpad):
    Bb, H, T = x_ref.shape
    E = wp_ref.shape[1]
    Hp = (H + 7) // 8 * 8
    x3 = x_ref[...].astype(jnp.bfloat16)
    if Hp > H:
        x3 = jnp.concatenate(
            [x3, jnp.zeros((Bb, Hp - H, T), jnp.bfloat16)], axis=1)
    x2 = x3.reshape(Bb * Hp, T)
    if tpad > 0:
        x2 = jnp.concatenate(
            [x2, jnp.zeros((Bb * Hp, tpad), jnp.bfloat16)], axis=1)
    def stage1(w0):
        t0 = 5 * w0
        xs = x2[:, t0:t0 + rw]
        y = jnp.dot(xs, wt_ref[...],
                    preferred_element_type=jnp.float32)
        yb = y.astype(jnp.bfloat16)
        yb = (a1_ref[...].astype(jnp.bfloat16) * yb
              + c1_ref[...].astype(jnp.bfloat16))
        yb = _elu(yb)
        return jnp.swapaxes(yb.reshape(Bb, Hp, C * tw), 1, 2)

    def stage2(q, w0):
        hacc = jnp.zeros((Bb * tw, E), jnp.float32)
        for c in range(C):
            qc = q[:, c * tw:(c + 1) * tw, :].reshape(Bb * tw, Hp)
            hacc = jnp.dot(qc, w2_ref[c],
                           preferred_element_type=jnp.float32) + hacc
        h = a2_ref[...] * hacc + c2_ref[...]
        h = _elu(h).astype(jnp.bfloat16)
        out = jnp.dot(h, wp_ref[...],
                      preferred_element_type=jnp.float32) + bp_ref[...]
        o_ref[:, w0:w0 + tw, :] = out.reshape(Bb, tw, E)

    q_prev, w_prev = stage1(w0s[0]), w0s[0]
    for w0 in w0s[1:]:
        q_cur = stage1(w0)
        stage2(q_prev, w_prev)
        q_prev, w_prev = q_cur, w0
    stage2(q_prev, w_prev)


def kernel(x, w1, b1, g1, be1, m1, v1, w2, b2, g2, be2, m2, v2, wp, bp):
    B, H, T = x.shape
    C, K1, PK, PS, eps = 40, 25, 51, 5, 1e-5
    E = wp.shape[0]
    Wo = T - K1 + 1
    Wp = (Wo - PK) // PS + 1
    BW = PK + K1 - 1

    cums = jnp.concatenate(
        [jnp.zeros((C, 1), jnp.float32), jnp.cumsum(w1, axis=1)], axis=1)
    d = jnp.arange(BW)
    hi = jnp.minimum(d, K1 - 1) + 1
    lo = jnp.maximum(d - (PK - 1), 0)
    band = ((cums[:, hi] - cums[:, lo]) / PK).T

    tw = min(64, Wp)
    rw = BW + PS * (tw - 1)
    r = jnp.arange(rw)[:, None]
    wr = jnp.arange(tw)[None, :]
    dd = r - PS * wr
    valid = (dd >= 0) & (dd < BW)
    wt = jnp.where(valid[:, :, None], band[jnp.clip(dd, 0, BW - 1)], 0.0)
    wt = jnp.transpose(wt, (0, 2, 1)).reshape(rw, C * tw).astype(jnp.bfloat16)

    a1 = g1 / jnp.sqrt(v1 + eps)
    c1 = be1 + a1 * (b1 - m1)
    a2 = g2 / jnp.sqrt(v2 + eps)
    c2 = be2 + a2 * (b2 - m2)
    a1b = jnp.repeat(a1, tw)[None, :].astype(jnp.float32)
    c1b = jnp.repeat(c1, tw)[None, :].astype(jnp.float32)
    a2r = a2[None, :].astype(jnp.float32)
    c2r = c2[None, :].astype(jnp.float32)
    Hp = (H + 7) // 8 * 8
    w2hc = jnp.pad(jnp.transpose(w2, (1, 2, 0)),
                   ((0, 0), (0, Hp - H), (0, 0))).astype(jnp.bfloat16)
    wpt = jnp.transpose(wp).astype(jnp.bfloat16)
    bpr = bp[None, :].astype(jnp.float32)

    Bb = next(dv for dv in (8, 4, 2, 1) if B % dv == 0)
    nt = -(-Wp // tw)
    Wpad = nt * tw
    w0s = tuple(range(0, Wpad, tw))
    tpad = max(0, PS * w0s[-1] + rw - T)

    kern = functools.partial(_fused_kernel, w0s=w0s, tw=tw, rw=rw, C=C,
                             tpad=tpad)
    flops = 2 * B * nt * (H * rw * tw * C + tw * H * C * E + tw * C * E)
    out = pl.pallas_call(
        kern,
        out_shape=jax.ShapeDtypeStruct((B, Wpad, E), jnp.float32),
        grid=(B // Bb,),
        in_specs=[
            pl.BlockSpec((Bb, H, T), lambda i: (i, 0, 0)),
            pl.BlockSpec((rw, C * tw), lambda i: (0, 0)),
            pl.BlockSpec((1, C * tw), lambda i: (0, 0)),
            pl.BlockSpec((1, C * tw), lambda i: (0, 0)),
            pl.BlockSpec((C, Hp, E), lambda i: (0, 0, 0)),
            pl.BlockSpec((1, E), lambda i: (0, 0)),
            pl.BlockSpec((1, E), lambda i: (0, 0)),
            pl.BlockSpec((C, E), lambda i: (0, 0)),
            pl.BlockSpec((1, E), lambda i: (0, 0)),
        ],
        out_specs=pl.BlockSpec((Bb, Wpad, E), lambda i: (i, 0, 0)),
        compiler_params=pltpu.CompilerParams(
            dimension_semantics=("parallel",),
            vmem_limit_bytes=64 * 1024 * 1024,
        ),
        cost_estimate=pl.CostEstimate(
            flops=int(flops),
            transcendentals=int(B * nt * tw * C * (H + 1)),
            bytes_accessed=int(4 * B * H * T + 4 * B * Wpad * E),
        ),
    )(x, wt, a1b, c1b, w2hc, a2r, c2r, wpt, bpr)
    return out[:, :Wp, :]


# speedup vs baseline: 10.8994x; 1.2005x over previous
import functools

import jax
import jax.numpy as jnp
from jax.experimental import pallas as pl
from jax.experimental.pallas import tpu as pltpu


def _elu(v):
    return jnp.where(v > 0, v, jnp.exp(jnp.minimum(v, 0.0)) - 1.0)


def _fused_kernel(x_ref, wt_ref, a1_ref, c1_ref, w2_ref, a2_ref, c2_ref,
                  wp_ref, bp_ref, o_ref, *, w0s, tw, rw, C, tpad):
    Bb, H, T = x_ref.shape
    E = wp_ref.shape[1]
    Hp = (H + 7) // 8 * 8
    x3 = x_ref[...].astype(jnp.bfloat16)
    if Hp > H:
        x3 = jnp.concatenate(
            [x3, jnp.zeros((Bb, Hp - H, T), jnp.bfloat16)], axis=1)
    x2 = x3.reshape(Bb * Hp, T)
    if tpad > 0:
        x2 = jnp.concatenate(
            [x2, jnp.zeros((Bb * Hp, tpad), jnp.bfloat16)], axis=1)
    def stage1(w0):
        t0 = 5 * w0
        xs = x2[:, t0:t0 + rw]
        y = jnp.dot(xs, wt_ref[...],
                    preferred_element_type=jnp.float32)
        yb = y.astype(jnp.bfloat16)
        yb = (a1_ref[...].astype(jnp.bfloat16) * yb
              + c1_ref[...].astype(jnp.bfloat16))
        yb = _elu(yb)
        return jnp.swapaxes(yb.reshape(Bb, Hp, C * tw), 1, 2)

    def stage2(q, w0):
        hacc = jnp.zeros((Bb * tw, E), jnp.float32)
        for c in range(C):
            qc = q[:, c * tw:(c + 1) * tw, :].reshape(Bb * tw, Hp)
            hacc = jnp.dot(qc, w2_ref[c],
                           preferred_element_type=jnp.float32) + hacc
        h = a2_ref[...] * hacc + c2_ref[...]
        h = _elu(h).astype(jnp.bfloat16)
        out = jnp.dot(h, wp_ref[...],
                      preferred_element_type=jnp.float32) + bp_ref[...]
        o_ref[:, w0:w0 + tw, :] = out.reshape(Bb, tw, E)

    q_prev, w_prev = stage1(w0s[0]), w0s[0]
    for w0 in w0s[1:]:
        q_cur = stage1(w0)
        stage2(q_prev, w_prev)
        q_prev, w_prev = q_cur, w0
    stage2(q_prev, w_prev)


def kernel(x, w1, b1, g1, be1, m1, v1, w2, b2, g2, be2, m2, v2, wp, bp):
    B, H, T = x.shape
    C, K1, PK, PS, eps = 40, 25, 51, 5, 1e-5
    E = wp.shape[0]
    Wo = T - K1 + 1
    Wp = (Wo - PK) // PS + 1
    BW = PK + K1 - 1

    tw = min(64, Wp)
    rw = BW + PS * (tw - 1)
    jj = jnp.arange(K1)[:, None, None]
    wrr = jnp.arange(tw)[None, :, None]
    rr = jnp.arange(rw)[None, None, :]
    dd = rr - PS * wrr
    msel = ((dd >= jj) & (dd <= jj + PK - 1)).astype(jnp.float32) / PK
    wt = jnp.dot(w1, msel.reshape(K1, tw * rw))
    wt = (wt.reshape(C, tw, rw).transpose(2, 0, 1)
            .reshape(rw, C * tw).astype(jnp.bfloat16))

    a1 = g1 / jnp.sqrt(v1 + eps)
    c1 = be1 + a1 * (b1 - m1)
    a2 = g2 / jnp.sqrt(v2 + eps)
    c2 = be2 + a2 * (b2 - m2)
    a1b = jnp.repeat(a1, tw)[None, :].astype(jnp.float32)
    c1b = jnp.repeat(c1, tw)[None, :].astype(jnp.float32)
    a2r = a2[None, :].astype(jnp.float32)
    c2r = c2[None, :].astype(jnp.float32)
    Hp = (H + 7) // 8 * 8
    w2hc = jnp.pad(jnp.transpose(w2, (1, 2, 0)),
                   ((0, 0), (0, Hp - H), (0, 0))).astype(jnp.bfloat16)
    wpt = jnp.transpose(wp).astype(jnp.bfloat16)
    bpr = bp[None, :].astype(jnp.float32)

    Bb = next(dv for dv in (8, 4, 2, 1) if B % dv == 0)
    nt = -(-Wp // tw)
    Wpad = nt * tw
    w0s = tuple(range(0, Wpad, tw))
    tpad = max(0, PS * w0s[-1] + rw - T)

    kern = functools.partial(_fused_kernel, w0s=w0s, tw=tw, rw=rw, C=C,
                             tpad=tpad)
    flops = 2 * B * nt * (H * rw * tw * C + tw * H * C * E + tw * C * E)
    out = pl.pallas_call(
        kern,
        out_shape=jax.ShapeDtypeStruct((B, Wpad, E), jnp.float32),
        grid=(B // Bb,),
        in_specs=[
            pl.BlockSpec((Bb, H, T), lambda i: (i, 0, 0)),
            pl.BlockSpec((rw, C * tw), lambda i: (0, 0)),
            pl.BlockSpec((1, C * tw), lambda i: (0, 0)),
            pl.BlockSpec((1, C * tw), lambda i: (0, 0)),
            pl.BlockSpec((C, Hp, E), lambda i: (0, 0, 0)),
            pl.BlockSpec((1, E), lambda i: (0, 0)),
            pl.BlockSpec((1, E), lambda i: (0, 0)),
            pl.BlockSpec((C, E), lambda i: (0, 0)),
            pl.BlockSpec((1, E), lambda i: (0, 0)),
        ],
        out_specs=pl.BlockSpec((Bb, Wpad, E), lambda i: (i, 0, 0)),
        compiler_params=pltpu.CompilerParams(
            dimension_semantics=("parallel",),
            vmem_limit_bytes=64 * 1024 * 1024,
        ),
        cost_estimate=pl.CostEstimate(
            flops=int(flops),
            transcendentals=int(B * nt * tw * C * (H + 1)),
            bytes_accessed=int(4 * B * H * T + 4 * B * Wpad * E),
        ),
    )(x, wt, a1b, c1b, w2hc, a2r, c2r, wpt, bpr)
    return out[:, :Wp, :]


# speedup vs baseline: 11.0879x; 1.0173x over previous
import functools

import jax
import jax.numpy as jnp
from jax.experimental import pallas as pl
from jax.experimental.pallas import tpu as pltpu


def _elu(v):
    return jnp.where(v > 0, v, jnp.exp(jnp.minimum(v, 0.0)) - 1.0)


def _fused_kernel(x_ref, wt_ref, a1_ref, c1_ref, w2_ref, a2_ref, c2_ref,
                  wp_ref, bp_ref, o_ref, *, w0s, tw, rw, C, tpad):
    Bb, H, T = x_ref.shape
    E = wp_ref.shape[1]
    Hp = (H + 7) // 8 * 8
    x3 = x_ref[...].astype(jnp.bfloat16)
    if Hp > H:
        x3 = jnp.concatenate(
            [x3, jnp.zeros((Bb, Hp - H, T), jnp.bfloat16)], axis=1)
    x2 = x3.reshape(Bb * Hp, T)
    if tpad > 0:
        x2 = jnp.concatenate(
            [x2, jnp.zeros((Bb * Hp, tpad), jnp.bfloat16)], axis=1)
    def stage1(w0):
        t0 = 5 * w0
        xs = x2[:, t0:t0 + rw]
        y = jnp.dot(xs, wt_ref[...],
                    preferred_element_type=jnp.float32)
        yb = y.astype(jnp.bfloat16)
        yb = (a1_ref[...].astype(jnp.bfloat16) * yb
              + c1_ref[...].astype(jnp.bfloat16))
        yb = _elu(yb)
        return jnp.swapaxes(yb.reshape(Bb, Hp, C * tw), 1, 2)

    def stage2(q, w0):
        hacc = jnp.zeros((Bb * tw, E), jnp.float32)
        for c in range(C):
            qc = q[:, c * tw:(c + 1) * tw, :].reshape(Bb * tw, Hp)
            hacc = jnp.dot(qc, w2_ref[c],
                           preferred_element_type=jnp.float32) + hacc
        h = a2_ref[...] * hacc + c2_ref[...]
        h = _elu(h).astype(jnp.bfloat16)
        out = jnp.dot(h, wp_ref[...],
                      preferred_element_type=jnp.float32) + bp_ref[...]
        o_ref[:, w0:w0 + tw, :] = out.reshape(Bb, tw, E)

    q_prev, w_prev = stage1(w0s[0]), w0s[0]
    for w0 in w0s[1:]:
        q_cur = stage1(w0)
        stage2(q_prev, w_prev)
        q_prev, w_prev = q_cur, w0
    stage2(q_prev, w_prev)


def kernel(x, w1, b1, g1, be1, m1, v1, w2, b2, g2, be2, m2, v2, wp, bp):
    B, H, T = x.shape
    C, K1, PK, PS, eps = 40, 25, 51, 5, 1e-5
    E = wp.shape[0]
    Wo = T - K1 + 1
    Wp = (Wo - PK) // PS + 1
    BW = PK + K1 - 1

    tw = min(64, Wp)
    rw = BW + PS * (tw - 1)
    jj = jnp.arange(K1)[:, None, None]
    wrr = jnp.arange(tw)[None, :, None]
    rr = jnp.arange(rw)[None, None, :]
    dd = rr - PS * wrr
    msel = ((dd >= jj) & (dd <= jj + PK - 1)).astype(jnp.float32) / PK
    wt = jnp.dot(w1, msel.reshape(K1, tw * rw))
    wt = (wt.reshape(C, tw, rw).transpose(2, 0, 1)
            .reshape(rw, C * tw).astype(jnp.bfloat16))

    a1 = g1 / jnp.sqrt(v1 + eps)
    c1 = be1 + a1 * (b1 - m1)
    a2 = g2 / jnp.sqrt(v2 + eps)
    c2 = be2 + a2 * (b2 - m2)
    a1b = jnp.repeat(a1, tw)[None, :].astype(jnp.float32)
    c1b = jnp.repeat(c1, tw)[None, :].astype(jnp.float32)
    a2r = a2[None, :].astype(jnp.float32)
    c2r = c2[None, :].astype(jnp.float32)
    Hp = (H + 7) // 8 * 8
    w2hc = jnp.pad(jnp.transpose(w2, (1, 2, 0)),
                   ((0, 0), (0, Hp - H), (0, 0))).astype(jnp.bfloat16)
    wpt = jnp.transpose(wp).astype(jnp.bfloat16)
    bpr = bp[None, :].astype(jnp.float32)

    Bb = next(dv for dv in (16, 8, 4, 2, 1) if B % dv == 0)
    nt = -(-Wp // tw)
    Wpad = nt * tw
    w0s = tuple(range(0, Wpad, tw))
    tpad = max(0, PS * w0s[-1] + rw - T)

    kern = functools.partial(_fused_kernel, w0s=w0s, tw=tw, rw=rw, C=C,
                             tpad=tpad)
    flops = 2 * B * nt * (H * rw * tw * C + tw * H * C * E + tw * C * E)
    out = pl.pallas_call(
        kern,
        out_shape=jax.ShapeDtypeStruct((B, Wpad, E), jnp.float32),
        grid=(B // Bb,),
        in_specs=[
            pl.BlockSpec((Bb, H, T), lambda i: (i, 0, 0)),
            pl.BlockSpec((rw, C * tw), lambda i: (0, 0)),
            pl.BlockSpec((1, C * tw), lambda i: (0, 0)),
            pl.BlockSpec((1, C * tw), lambda i: (0, 0)),
            pl.BlockSpec((C, Hp, E), lambda i: (0, 0, 0)),
            pl.BlockSpec((1, E), lambda i: (0, 0)),
            pl.BlockSpec((1, E), lambda i: (0, 0)),
            pl.BlockSpec((C, E), lambda i: (0, 0)),
            pl.BlockSpec((1, E), lambda i: (0, 0)),
        ],
        out_specs=pl.BlockSpec((Bb, Wpad, E), lambda i: (i, 0, 0)),
        compiler_params=pltpu.CompilerParams(
            dimension_semantics=("parallel",),
            vmem_limit_bytes=64 * 1024 * 1024,
        ),
        cost_estimate=pl.CostEstimate(
            flops=int(flops),
            transcendentals=int(B * nt * tw * C * (H + 1)),
            bytes_accessed=int(4 * B * H * T + 4 * B * Wpad * E),
        ),
    )(x, wt, a1b, c1b, w2hc, a2r, c2r, wpt, bpr)
    return out[:, :Wp, :]
